# baseline (speedup 1.0000x reference)
import functools

import jax
import jax.numpy as jnp
from jax import lax
from jax.experimental import pallas as pl
from jax.experimental.pallas import tpu as pltpu
from jax.experimental.pallas import tpu_sc as plsc

NUM_ENTITIES = 1000000
NUM_RELATIONS = 1000
EMBED_DIM = 64
HALF_DIM = EMBED_DIM // 2
BATCH = 16384

NC = 2
NS = 16
NW = NC * NS
B_PER_W = BATCH // NW
PHASE = 16
N_PHASES = B_PER_W // PHASE
ROT_ROW = 2 * EMBED_DIM

N_BLOCKS = (NUM_ENTITIES + 127) // 128
FULL_ROUNDS = N_BLOCKS // NW
EXTRA_BLOCKS = N_BLOCKS - FULL_ROUNDS * NW
PAIR_ROWS = 64 * N_BLOCKS


def _rot_body(p_ref, o_ref):
    ph = p_ref[...]
    z = jnp.zeros_like(ph)
    o_ref[...] = jnp.concatenate([jnp.cos(ph), jnp.sin(ph), z, z], axis=-1)


def _rsqrt(x):
    i = plsc.bitcast(x, jnp.int32)
    i = jnp.int32(0x5F3759DF) - lax.shift_right_logical(i, 1)
    y = plsc.bitcast(i, jnp.float32)
    half = jnp.float32(0.5) * x
    for _ in range(2):
        y = y * (jnp.float32(1.5) - half * y * y)
    return y


def _relayout_body(ent_t_hbm, out_hbm, ibuf, obuf, isem_a, isem_b, osem):
    wid = lax.axis_index("s") * NC + lax.axis_index("c")
    d0 = lax.iota(jnp.int32, 16)

    def fire_in(j, slot_base, sem):
        b = wid + NW * j
        pltpu.async_copy(ent_t_hbm.at[:, pl.ds(b * 128, 128)],
                         ibuf.at[pl.ds(slot_base, EMBED_DIM)], sem)

    def drain_in(sem):
        pltpu.make_async_copy(ent_t_hbm.at[:, pl.ds(0, 128)],
                              ibuf.at[pl.ds(0, EMBED_DIM)], sem).wait()

    def transpose_block(in_base, out_base):
        def row_body(r, _):
            e0 = jnp.full((16,), 2 * r, jnp.int32)
            e1 = e0 + 1
            for c in range(8):
                dv = d0 + (16 * (c % 4) + in_base)
                v = plsc.load_gather(ibuf, [dv, e0 if c < 4 else e1])
                obuf[out_base + r, pl.ds(16 * c, 16)] = v
            return 0

        lax.fori_loop(0, EMBED_DIM, row_body, 0)

    def fire_out(j, out_base):
        b = wid + NW * j
        pltpu.async_copy(obuf.at[pl.ds(out_base, EMBED_DIM)],
                         out_hbm.at[pl.ds(b * 64, EMBED_DIM)], osem)

    def drain_out():
        pltpu.make_async_copy(ent_t_hbm.at[:, pl.ds(0, 128)],
                              obuf.at[pl.ds(0, EMBED_DIM)], osem).wait()

    fire_in(0, 0, isem_a)

    def round_body(j, _):
        even = j % 2 == 0

        @pl.when(j + 1 < FULL_ROUNDS)
        def _():
            @pl.when(even)
            def _():
                fire_in(j + 1, EMBED_DIM, isem_b)

            @pl.when(jnp.logical_not(even))
            def _():
                fire_in(j + 1, 0, isem_a)

        @pl.when(j >= 2)
        def _():
            drain_out()

        @pl.when(even)
        def _():
            drain_in(isem_a)
            transpose_block(0, 0)
            fire_out(j, 0)

        @pl.when(jnp.logical_not(even))
        def _():
            drain_in(isem_b)
            transpose_block(EMBED_DIM, EMBED_DIM)
            fire_out(j, EMBED_DIM)

        return 0

    lax.fori_loop(0, FULL_ROUNDS, round_body, 0)
    drain_out()
    drain_out()

    @pl.when(wid < EXTRA_BLOCKS)
    def _():
        b = FULL_ROUNDS * NW + wid
        pltpu.async_copy(ent_t_hbm.at[:, pl.ds(b * 128, 128)],
                         ibuf.at[pl.ds(0, EMBED_DIM)], isem_a).wait()
        transpose_block(0, 0)
        pltpu.async_copy(obuf.at[pl.ds(0, EMBED_DIM)],
                         out_hbm.at[pl.ds(b * 64, EMBED_DIM)], osem).wait()


def _score_body(pair_hbm, rot_hbm, heads_hbm, tails_hbm, rels_hbm, out_hbm,
                idx_h, idx_t, idx_r, h2, t2, rr, out_v, sem_a, sem_b,
                rsem_a, rsem_b):
    wid = lax.axis_index("s") * NC + lax.axis_index("c")
    base = wid * B_PER_W

    pltpu.sync_copy(heads_hbm.at[pl.ds(base, B_PER_W)], idx_h)
    pltpu.sync_copy(tails_hbm.at[pl.ds(base, B_PER_W)], idx_t)
    pltpu.sync_copy(rels_hbm.at[wid], idx_r)

    eps = jnp.float32(1e-12)
    lane = lax.iota(jnp.int32, 16)
    zeros = jnp.zeros((16,), jnp.float32)
    one = jnp.full((16,), 1, jnp.int32)

    def fire(ph, slot, sem, rsem):
        s0 = slot * PHASE
        pltpu.async_copy(rot_hbm.at[idx_r.at[ph]],
                         rr.at[pl.ds(s0, PHASE)], rsem)
        hp = lax.shift_right_logical(idx_h[pl.ds(ph * PHASE, PHASE)], 1)
        tp = lax.shift_right_logical(idx_t[pl.ds(ph * PHASE, PHASE)], 1)
        pltpu.async_copy(pair_hbm.at[hp], h2.at[pl.ds(s0, PHASE)], sem)
        pltpu.async_copy(pair_hbm.at[tp], t2.at[pl.ds(s0, PHASE)], sem)

    def drain(sem, rsem):
        pltpu.make_async_copy(rot_hbm.at[pl.ds(0, PHASE)],
                              rr.at[pl.ds(0, PHASE)], rsem).wait()
        pltpu.make_async_copy(pair_hbm.at[pl.ds(0, PHASE)],
                              h2.at[pl.ds(0, PHASE)], sem).wait()
        pltpu.make_async_copy(pair_hbm.at[pl.ds(0, PHASE)],
                              t2.at[pl.ds(0, PHASE)], sem).wait()

    def compute(ph, slot):
        s0 = slot * PHASE
        ph_h = idx_h[pl.ds(ph * PHASE, 16)] & one
        ph_t = idx_t[pl.ds(ph * PHASE, 16)] & one
        score = zeros
        for k in range(16):
            i = s0 + k
            mh = jnp.full((16,), ph_h[k], jnp.int32) == one
            mt = jnp.full((16,), ph_t[k], jnp.int32) == one
            acc = None
            for off in (0, 16):
                h_re = jnp.where(mh, h2[i, pl.ds(EMBED_DIM + off, 16)],
                                 h2[i, pl.ds(off, 16)])
                h_im = jnp.where(mh, h2[i, pl.ds(EMBED_DIM + HALF_DIM + off, 16)],
                                 h2[i, pl.ds(HALF_DIM + off, 16)])
                t_re = jnp.where(mt, t2[i, pl.ds(EMBED_DIM + off, 16)],
                                 t2[i, pl.ds(off, 16)])
                t_im = jnp.where(mt, t2[i, pl.ds(EMBED_DIM + HALF_DIM + off, 16)],
                                 t2[i, pl.ds(HALF_DIM + off, 16)])
                c_re = rr[i, pl.ds(off, 16)]
                c_im = rr[i, pl.ds(HALF_DIM + off, 16)]
                diff_re = h_re * c_re - h_im * c_im - t_re
                diff_im = h_re * c_im + h_im * c_re - t_im
                sq = diff_re * diff_re + diff_im * diff_im + eps
                mag = sq * _rsqrt(sq)
                acc = mag if acc is None else acc + mag
            s = jnp.full((16,), jnp.sum(acc), jnp.float32)
            score = jnp.where(lane == k, s, score)
        out_v[pl.ds(ph * PHASE, 16)] = -score

    fire(0, 0, sem_a, rsem_a)

    def phase_body(ph, _):
        even = ph % 2 == 0

        @pl.when(ph + 1 < N_PHASES)
        def _():
            @pl.when(even)
            def _():
                fire(ph + 1, 1, sem_b, rsem_b)

            @pl.when(jnp.logical_not(even))
            def _():
                fire(ph + 1, 0, sem_a, rsem_a)

        @pl.when(even)
        def _():
            drain(sem_a, rsem_a)
            compute(ph, 0)

        @pl.when(jnp.logical_not(even))
        def _():
            drain(sem_b, rsem_b)
            compute(ph, 1)

        return 0

    lax.fori_loop(0, N_PHASES, phase_body, 0)

    pltpu.sync_copy(out_v, out_hbm.at[pl.ds(base, B_PER_W)])


def _sc_params():
    return dict(
        mesh=plsc.VectorSubcoreMesh(core_axis_name="c", subcore_axis_name="s",
                                    num_cores=NC, num_subcores=NS),
        compiler_params=pltpu.CompilerParams(needs_layout_passes=False,
                                             use_tc_tiling_on_sc=True),
    )


@functools.lru_cache(maxsize=1)
def _relayout_call():
    return pl.kernel(
        _relayout_body,
        out_type=jax.ShapeDtypeStruct((PAIR_ROWS, 2 * EMBED_DIM), jnp.float32),
        scratch_types=[
            pltpu.VMEM((2 * EMBED_DIM, 128), jnp.float32),
            pltpu.VMEM((2 * EMBED_DIM, 128), jnp.float32),
            pltpu.SemaphoreType.DMA,
            pltpu.SemaphoreType.DMA,
            pltpu.SemaphoreType.DMA,
        ],
        **_sc_params(),
    )


@functools.lru_cache(maxsize=1)
def _score_call():
    return pl.kernel(
        _score_body,
        out_type=jax.ShapeDtypeStruct((BATCH,), jnp.float32),
        scratch_types=[
            pltpu.VMEM((B_PER_W,), jnp.int32),
            pltpu.VMEM((B_PER_W,), jnp.int32),
            pltpu.VMEM((N_PHASES, PHASE), jnp.int32),
            pltpu.VMEM((2 * PHASE, 2 * EMBED_DIM), jnp.float32),
            pltpu.VMEM((2 * PHASE, 2 * EMBED_DIM), jnp.float32),
            pltpu.VMEM((2 * PHASE, ROT_ROW), jnp.float32),
            pltpu.VMEM((B_PER_W,), jnp.float32),
            pltpu.SemaphoreType.DMA,
            pltpu.SemaphoreType.DMA,
            pltpu.SemaphoreType.DMA,
            pltpu.SemaphoreType.DMA,
        ],
        **_sc_params(),
    )


_rot_call = pl.pallas_call(
    _rot_body,
    out_shape=jax.ShapeDtypeStruct((NUM_RELATIONS, ROT_ROW), jnp.float32),
)


@jax.jit
def kernel(entity_emb, heads, relations, tails, relation_phase_weight):
    rot = _rot_call(relation_phase_weight)
    pair_table = _relayout_call()(entity_emb.T)
    rels3 = relations.astype(jnp.int32).reshape(NW, N_PHASES, PHASE)
    return _score_call()(pair_table, rot, heads.astype(jnp.int32),
                         tails.astype(jnp.int32), rels3)


# speedup vs baseline: 5.2465x; 5.2465x over previous
import functools

import jax
import jax.numpy as jnp
from jax import lax
from jax.experimental import pallas as pl
from jax.experimental.pallas import tpu as pltpu
from jax.experimental.pallas import tpu_sc as plsc

NUM_ENTITIES = 1000000
NUM_RELATIONS = 1000
EMBED_DIM = 64
HALF_DIM = EMBED_DIM // 2
BATCH = 16384

NC = 2
NS = 16
NW = NC * NS
B_PER_W = BATCH // NW
PHASE = 16
N_PHASES = B_PER_W // PHASE
ROT_ROW = 2 * EMBED_DIM


def _rot_body(p_ref, o_ref):
    ph = p_ref[...]
    z = jnp.zeros_like(ph)
    o_ref[...] = jnp.concatenate([jnp.cos(ph), jnp.sin(ph), z, z], axis=-1)


def _rsqrt(x):
    i = plsc.bitcast(x, jnp.int32)
    i = jnp.int32(0x5F3759DF) - lax.shift_right_logical(i, 1)
    y = plsc.bitcast(i, jnp.float32)
    half = jnp.float32(0.5) * x
    for _ in range(2):
        y = y * (jnp.float32(1.5) - half * y * y)
    return y


def _sc_body(ent_hbm, rot_hbm, heads_hbm, tails_hbm, rels_hbm, out_hbm,
             idx_h, idx_t, idx_r, h8, t8, rr, out_v, sem_a, sem_b,
             rsem_a, rsem_b):
    wid = lax.axis_index("s") * NC + lax.axis_index("c")
    base = wid * B_PER_W

    pltpu.sync_copy(heads_hbm.at[pl.ds(base, B_PER_W)], idx_h)
    pltpu.sync_copy(tails_hbm.at[pl.ds(base, B_PER_W)], idx_t)
    pltpu.sync_copy(rels_hbm.at[wid], idx_r)

    eps = jnp.float32(1e-12)
    lane = lax.iota(jnp.int32, 16)
    zeros = jnp.zeros((16,), jnp.float32)
    seven = jnp.full((16,), 7, jnp.int32)

    def fire(ph, slot, sem, rsem):
        s0 = slot * PHASE
        pltpu.async_copy(rot_hbm.at[idx_r.at[ph]],
                         rr.at[pl.ds(s0, PHASE)], rsem)
        hv = idx_h[pl.ds(ph * PHASE, PHASE)]
        tv = idx_t[pl.ds(ph * PHASE, PHASE)]
        hb = lax.shift_right_logical(hv, 3)
        tb = lax.shift_right_logical(tv, 3)
        for k in range(PHASE):
            pltpu.async_copy(ent_hbm.at[hb[k]], h8.at[s0 + k], sem)
            pltpu.async_copy(ent_hbm.at[tb[k]], t8.at[s0 + k], sem)

    def drain(sem, rsem):
        pltpu.make_async_copy(rot_hbm.at[pl.ds(0, PHASE)],
                              rr.at[pl.ds(0, PHASE)], rsem).wait()
        pltpu.make_async_copy(ent_hbm.at[pl.ds(0, PHASE)],
                              h8.at[pl.ds(0, PHASE)], sem).wait()
        pltpu.make_async_copy(ent_hbm.at[pl.ds(0, PHASE)],
                              t8.at[pl.ds(0, PHASE)], sem).wait()

    def compute(ph, slot):
        s0 = slot * PHASE
        rh = idx_h[pl.ds(ph * PHASE, 16)] & seven
        rt = idx_t[pl.ds(ph * PHASE, 16)] & seven
        score = zeros
        for k in range(16):
            i = s0 + k
            acc = None
            for off in (0, 16):
                h_re = h8[i, rh[k], pl.ds(off, 16)]
                h_im = h8[i, rh[k], pl.ds(HALF_DIM + off, 16)]
                t_re = t8[i, rt[k], pl.ds(off, 16)]
                t_im = t8[i, rt[k], pl.ds(HALF_DIM + off, 16)]
                c_re = rr[i, pl.ds(off, 16)]
                c_im = rr[i, pl.ds(HALF_DIM + off, 16)]
                diff_re = h_re * c_re - h_im * c_im - t_re
                diff_im = h_re * c_im + h_im * c_re - t_im
                sq = diff_re * diff_re + diff_im * diff_im + eps
                mag = sq * _rsqrt(sq)
                acc = mag if acc is None else acc + mag
            s = jnp.full((16,), jnp.sum(acc), jnp.float32)
            score = jnp.where(lane == k, s, score)
        out_v[pl.ds(ph * PHASE, 16)] = -score

    fire(0, 0, sem_a, rsem_a)

    def phase_body(ph, _):
        even = ph % 2 == 0

        @pl.when(ph + 1 < N_PHASES)
        def _():
            @pl.when(even)
            def _():
                fire(ph + 1, 1, sem_b, rsem_b)

            @pl.when(jnp.logical_not(even))
            def _():
                fire(ph + 1, 0, sem_a, rsem_a)

        @pl.when(even)
        def _():
            drain(sem_a, rsem_a)
            compute(ph, 0)

        @pl.when(jnp.logical_not(even))
        def _():
            drain(sem_b, rsem_b)
            compute(ph, 1)

        return 0

    lax.fori_loop(0, N_PHASES, phase_body, 0)

    pltpu.sync_copy(out_v, out_hbm.at[pl.ds(base, B_PER_W)])


@functools.lru_cache(maxsize=1)
def _sc_call():
    return pl.kernel(
        _sc_body,
        out_type=jax.ShapeDtypeStruct((BATCH,), jnp.float32),
        mesh=plsc.VectorSubcoreMesh(core_axis_name="c", subcore_axis_name="s",
                                    num_cores=NC, num_subcores=NS),
        compiler_params=pltpu.CompilerParams(needs_layout_passes=False,
                                             use_tc_tiling_on_sc=True),
        scratch_types=[
            pltpu.VMEM((B_PER_W,), jnp.int32),
            pltpu.VMEM((B_PER_W,), jnp.int32),
            pltpu.VMEM((N_PHASES, PHASE), jnp.int32),
            pltpu.VMEM((2 * PHASE, 8, EMBED_DIM), jnp.float32),
            pltpu.VMEM((2 * PHASE, 8, EMBED_DIM), jnp.float32),
            pltpu.VMEM((2 * PHASE, ROT_ROW), jnp.float32),
            pltpu.VMEM((B_PER_W,), jnp.float32),
            pltpu.SemaphoreType.DMA,
            pltpu.SemaphoreType.DMA,
            pltpu.SemaphoreType.DMA,
            pltpu.SemaphoreType.DMA,
        ],
    )


_rot_call = pl.pallas_call(
    _rot_body,
    out_shape=jax.ShapeDtypeStruct((NUM_RELATIONS, ROT_ROW), jnp.float32),
)


@jax.jit
def kernel(entity_emb, heads, relations, tails, relation_phase_weight):
    rot = _rot_call(relation_phase_weight)
    rels3 = relations.astype(jnp.int32).reshape(NW, N_PHASES, PHASE)
    ent3 = entity_emb.reshape(NUM_ENTITIES // 8, 8, EMBED_DIM)
    return _sc_call()(ent3, rot, heads.astype(jnp.int32),
                      tails.astype(jnp.int32), rels3)
